# Initial kernel scaffold; baseline (speedup 1.0000x reference)
import jax
import jax.numpy as jnp
from jax.experimental import pallas as pl


def kernel(h, r, t, entity_emb, relation_emb):
    raise NotImplementedError("write your pallas kernel here")


# speedup vs baseline: 1.9231x; 1.9231x over previous
import functools

import jax
import jax.numpy as jnp
from jax import lax
from jax.experimental import pallas as pl
from jax.experimental.pallas import tpu as pltpu
from jax.experimental.pallas import tpu_sc as plsc

_NUM_ENTITIES = 100000
_NUM_RELATIONS = 1000
_D = 128
_BATCH = 16384

_L = 16
_NW = 32
_B_PER_W = _BATCH // _NW
_CHUNK = 128
_NCHUNK = _B_PER_W // _CHUNK


def _sc_transe(h_hbm, r_hbm, t_hbm, ent_hbm, rel_hbm, out_hbm,
               hi_v, ri_v, ti_v, he_v, re_v, te_v, sem):
    wid = lax.axis_index("s") * 2 + lax.axis_index("c")
    base = wid * _B_PER_W
    for c in range(_NCHUNK):
        off = base + c * _CHUNK
        pltpu.sync_copy(h_hbm.at[pl.ds(off, _CHUNK)], hi_v)
        pltpu.sync_copy(r_hbm.at[pl.ds(off, _CHUNK)], ri_v)
        pltpu.sync_copy(t_hbm.at[pl.ds(off, _CHUNK)], ti_v)
        cp_h = pltpu.async_copy(ent_hbm.at[hi_v], he_v, sem)
        cp_r = pltpu.async_copy(rel_hbm.at[ri_v], re_v, sem)
        cp_t = pltpu.async_copy(ent_hbm.at[ti_v], te_v, sem)
        cp_h.wait()
        cp_r.wait()
        cp_t.wait()

        def body(i, carry):
            for j in range(_D // _L):
                s = pl.ds(j * _L, _L)
                he_v[i, s] = he_v[i, s] + re_v[i, s] - te_v[i, s]
            return carry

        lax.fori_loop(0, _CHUNK, body, 0)
        pltpu.sync_copy(he_v, out_hbm.at[pl.ds(off, _CHUNK)])


def kernel(h, r, t, entity_emb, relation_emb):
    h = h.astype(jnp.int32)
    r = r.astype(jnp.int32)
    t = t.astype(jnp.int32)
    mesh = plsc.VectorSubcoreMesh(core_axis_name="c", subcore_axis_name="s")
    run = functools.partial(
        pl.kernel,
        mesh=mesh,
        out_type=jax.ShapeDtypeStruct((_BATCH, _D), jnp.float32),
        scratch_types=[
            pltpu.VMEM((_CHUNK,), jnp.int32),
            pltpu.VMEM((_CHUNK,), jnp.int32),
            pltpu.VMEM((_CHUNK,), jnp.int32),
            pltpu.VMEM((_CHUNK, _D), jnp.float32),
            pltpu.VMEM((_CHUNK, _D), jnp.float32),
            pltpu.VMEM((_CHUNK, _D), jnp.float32),
            pltpu.SemaphoreType.DMA,
        ],
    )(_sc_transe)
    return run(h, r, t, entity_emb, relation_emb)


# speedup vs baseline: 2.3898x; 1.2427x over previous
import functools

import jax
import jax.numpy as jnp
from jax import lax
from jax.experimental import pallas as pl
from jax.experimental.pallas import tpu as pltpu
from jax.experimental.pallas import tpu_sc as plsc

_D = 128
_BATCH = 16384

_L = 16
_NW = 32
_B_PER_W = _BATCH // _NW
_CHUNK = 128
_NCHUNK = _B_PER_W // _CHUNK


def _sc_transe(h_hbm, r_hbm, t_hbm, ent_hbm, rel_hbm, out_hbm,
               hi_v, ri_v, ti_v, he_v, re_v, te_v, gsem, wsem):
    wid = lax.axis_index("s") * 2 + lax.axis_index("c")
    base = wid * _B_PER_W
    pltpu.sync_copy(h_hbm.at[pl.ds(base, _B_PER_W)], hi_v)
    pltpu.sync_copy(r_hbm.at[pl.ds(base, _B_PER_W)], ri_v)
    pltpu.sync_copy(t_hbm.at[pl.ds(base, _B_PER_W)], ti_v)

    def gather(c, p):
        s = pl.ds(c * _CHUNK, _CHUNK)
        cps = (pltpu.async_copy(ent_hbm.at[hi_v.at[s]], he_v.at[p], gsem.at[p]),
               pltpu.async_copy(rel_hbm.at[ri_v.at[s]], re_v.at[p], gsem.at[p]),
               pltpu.async_copy(ent_hbm.at[ti_v.at[s]], te_v.at[p], gsem.at[p]))
        return cps

    pending = {0: gather(0, 0)}
    wb = {}
    for c in range(_NCHUNK):
        p = c & 1
        if c + 1 < _NCHUNK:
            q = 1 - p
            if c >= 1:
                wb.pop(c - 1)[1].wait()
            pending[c + 1] = gather(c + 1, q)
        for cp in pending.pop(c):
            cp.wait()

        def body(i, carry):
            for j in range(_D // _L):
                s = pl.ds(j * _L, _L)
                he_v[p, i, s] = he_v[p, i, s] + re_v[p, i, s] - te_v[p, i, s]
            return carry

        lax.fori_loop(0, _CHUNK, body, 0)
        wb[c] = (p, pltpu.async_copy(
            he_v.at[p], out_hbm.at[pl.ds(base + c * _CHUNK, _CHUNK)], wsem.at[p]))
    for c in sorted(wb):
        wb[c][1].wait()


def kernel(h, r, t, entity_emb, relation_emb):
    h = h.astype(jnp.int32)
    r = r.astype(jnp.int32)
    t = t.astype(jnp.int32)
    mesh = plsc.VectorSubcoreMesh(core_axis_name="c", subcore_axis_name="s")
    run = functools.partial(
        pl.kernel,
        mesh=mesh,
        out_type=jax.ShapeDtypeStruct((_BATCH, _D), jnp.float32),
        scratch_types=[
            pltpu.VMEM((_B_PER_W,), jnp.int32),
            pltpu.VMEM((_B_PER_W,), jnp.int32),
            pltpu.VMEM((_B_PER_W,), jnp.int32),
            pltpu.VMEM((2, _CHUNK, _D), jnp.float32),
            pltpu.VMEM((2, _CHUNK, _D), jnp.float32),
            pltpu.VMEM((2, _CHUNK, _D), jnp.float32),
            pltpu.SemaphoreType.DMA((2,)),
            pltpu.SemaphoreType.DMA((2,)),
        ],
    )(_sc_transe)
    return run(h, r, t, entity_emb, relation_emb)
